# baseline (speedup 1.0000x reference)
import functools

import jax
import jax.numpy as jnp
from jax import lax
from jax.experimental import pallas as pl
from jax.experimental.pallas import tpu as pltpu
from jax.experimental.pallas import tpu_sc as plsc

STEP = 0.01
NC = 2
NS = 16
NW = NC * NS
C = 128
ZRV = 400


def _node_pad(n):
    q = NS * 3200
    return ((n + q - 1) // q) * q


def _sc_degree(n, e):
    npad = _node_pad(n)
    nchunks = e // C
    rpt = npad // NS
    mesh = plsc.VectorSubcoreMesh(core_axis_name="c", subcore_axis_name="s")

    @functools.partial(
        pl.kernel,
        mesh=mesh,
        out_type=jax.ShapeDtypeStruct((NC, npad), jnp.float32),
        compiler_params=pltpu.CompilerParams(use_tc_tiling_on_sc=False),
        scratch_types=[
            pltpu.VMEM((C,), jnp.int32),
            pltpu.VMEM((C,), jnp.float32),
            pltpu.VMEM((ZRV,), jnp.float32),
            pltpu.VMEM_SHARED((npad,), jnp.float32),
        ],
    )
    def deg_kernel(src_hbm, ones_hbm, zv_hbm, out_hbm, srcb, onesb, vzb, acc):
        c = lax.axis_index("c")
        s = lax.axis_index("s")
        wid = c * NS + s
        zb = s * rpt
        pltpu.sync_copy(ones_hbm, onesb)
        pltpu.sync_copy(zv_hbm, vzb)
        for j in range(rpt // ZRV):
            pltpu.sync_copy(vzb, acc.at[pl.ds(zb + j * ZRV, ZRV)])
        plsc.subcore_barrier()

        nj = (nchunks + NW - 1 - wid) // NW

        def body(j, carry):
            off = (wid + j * NW) * C
            pltpu.sync_copy(src_hbm.at[pl.ds(off, C)], srcb)
            pltpu.sync_copy(onesb, acc.at[srcb], add=True)
            return carry

        lax.fori_loop(0, nj, body, 0)
        plsc.subcore_barrier()
        for j in range(rpt // ZRV):
            r0 = zb + j * ZRV
            pltpu.sync_copy(acc.at[pl.ds(r0, ZRV)], vzb)
            pltpu.sync_copy(vzb, out_hbm.at[c, pl.ds(r0, ZRV)])

    return deg_kernel


def _sc_push16v(n, e):
    npad = _node_pad(n)
    nchunks = e // C
    rpt = npad // NS
    mesh = plsc.VectorSubcoreMesh(core_axis_name="c", subcore_axis_name="s")

    @functools.partial(
        pl.kernel,
        mesh=mesh,
        out_type=[
            jax.ShapeDtypeStruct((NC, npad, 16), jnp.float32),
            jax.ShapeDtypeStruct((NC, npad), jnp.float32),
        ],
        compiler_params=pltpu.CompilerParams(use_tc_tiling_on_sc=False),
        scratch_types=[
            pltpu.VMEM((C,), jnp.int32),
            pltpu.VMEM((C,), jnp.int32),
            pltpu.VMEM((C,), jnp.int32),
            pltpu.VMEM((C,), jnp.int32),
            pltpu.VMEM((C, 16), jnp.float32),
            pltpu.VMEM((C, 16), jnp.float32),
            pltpu.VMEM((C,), jnp.float32),
            pltpu.VMEM((C,), jnp.float32),
            pltpu.VMEM((ZRV,), jnp.float32),
            pltpu.VMEM_SHARED((npad, 16), jnp.float32),
            pltpu.VMEM_SHARED((npad,), jnp.float32),
            pltpu.SemaphoreType.DMA,
            pltpu.SemaphoreType.DMA,
            pltpu.SemaphoreType.DMA,
            pltpu.SemaphoreType.DMA,
        ],
    )
    def push_kernel(wq_hbm, wv_hbm, src_hbm, dst_hbm, z16_hbm, zv_hbm,
                    out16_hbm, outv_hbm,
                    srcb0, srcb1, dstb0, dstb1, rows0, rows1, vrows0, vrows1,
                    vzb, acc16, accv, sem1, sem2, sem3, sem4):
        c = lax.axis_index("c")
        s = lax.axis_index("s")
        wid = c * NS + s
        zb = s * rpt
        pltpu.sync_copy(z16_hbm, rows0)
        pltpu.sync_copy(zv_hbm, vzb)
        for j in range(rpt // C):
            pltpu.sync_copy(rows0, acc16.at[pl.ds(zb + j * C, C)])
        for j in range(rpt // ZRV):
            pltpu.sync_copy(vzb, accv.at[pl.ds(zb + j * ZRV, ZRV)])
        plsc.subcore_barrier()

        nj = (nchunks + NW - 1 - wid) // NW

        def body(j, carry):
            off0 = (wid + (2 * j) * NW) * C
            off1 = (wid + (2 * j + 1) * NW) * C
            pltpu.sync_copy(src_hbm.at[pl.ds(off0, C)], srcb0)
            pltpu.sync_copy(dst_hbm.at[pl.ds(off0, C)], dstb0)
            g0 = pltpu.async_copy(wq_hbm.at[srcb0], rows0, sem1)
            gv0 = pltpu.async_copy(wv_hbm.at[srcb0], vrows0, sem3)
            pltpu.sync_copy(src_hbm.at[pl.ds(off1, C)], srcb1)
            pltpu.sync_copy(dst_hbm.at[pl.ds(off1, C)], dstb1)
            g1 = pltpu.async_copy(wq_hbm.at[srcb1], rows1, sem2)
            gv1 = pltpu.async_copy(wv_hbm.at[srcb1], vrows1, sem4)
            g0.wait()
            pltpu.sync_copy(rows0, acc16.at[dstb0], add=True)
            gv0.wait()
            pltpu.sync_copy(vrows0, accv.at[dstb0], add=True)
            g1.wait()
            pltpu.sync_copy(rows1, acc16.at[dstb1], add=True)
            gv1.wait()
            pltpu.sync_copy(vrows1, accv.at[dstb1], add=True)
            return carry

        lax.fori_loop(0, nj // 2, body, 0)

        @pl.when(nj % 2 == 1)
        def _tail():
            off0 = (wid + (nj - 1) * NW) * C
            pltpu.sync_copy(src_hbm.at[pl.ds(off0, C)], srcb0)
            pltpu.sync_copy(dst_hbm.at[pl.ds(off0, C)], dstb0)
            g0 = pltpu.async_copy(wq_hbm.at[srcb0], rows0, sem1)
            gv0 = pltpu.async_copy(wv_hbm.at[srcb0], vrows0, sem3)
            g0.wait()
            pltpu.sync_copy(rows0, acc16.at[dstb0], add=True)
            gv0.wait()
            pltpu.sync_copy(vrows0, accv.at[dstb0], add=True)

        plsc.subcore_barrier()
        for j in range(rpt // C):
            r0 = zb + j * C
            pltpu.sync_copy(acc16.at[pl.ds(r0, C)], rows0)
            pltpu.sync_copy(rows0, out16_hbm.at[c, pl.ds(r0, C)])
        for j in range(rpt // ZRV):
            r0 = zb + j * ZRV
            pltpu.sync_copy(accv.at[pl.ds(r0, ZRV)], vzb)
            pltpu.sync_copy(vzb, outv_hbm.at[c, pl.ds(r0, ZRV)])

    return push_kernel


def _sc_push16(n, e):
    npad = _node_pad(n)
    nchunks = e // C
    rpt = npad // NS
    mesh = plsc.VectorSubcoreMesh(core_axis_name="c", subcore_axis_name="s")

    @functools.partial(
        pl.kernel,
        mesh=mesh,
        out_type=jax.ShapeDtypeStruct((NC, npad, 16), jnp.float32),
        compiler_params=pltpu.CompilerParams(use_tc_tiling_on_sc=False),
        scratch_types=[
            pltpu.VMEM((C,), jnp.int32),
            pltpu.VMEM((C,), jnp.int32),
            pltpu.VMEM((C,), jnp.int32),
            pltpu.VMEM((C,), jnp.int32),
            pltpu.VMEM((C, 16), jnp.float32),
            pltpu.VMEM((C, 16), jnp.float32),
            pltpu.VMEM_SHARED((npad, 16), jnp.float32),
            pltpu.SemaphoreType.DMA,
            pltpu.SemaphoreType.DMA,
        ],
    )
    def push_kernel(wy_hbm, src_hbm, dst_hbm, z16_hbm, out16_hbm,
                    srcb0, srcb1, dstb0, dstb1, rows0, rows1, acc16,
                    sem1, sem2):
        c = lax.axis_index("c")
        s = lax.axis_index("s")
        wid = c * NS + s
        zb = s * rpt
        pltpu.sync_copy(z16_hbm, rows0)
        for j in range(rpt // C):
            pltpu.sync_copy(rows0, acc16.at[pl.ds(zb + j * C, C)])
        plsc.subcore_barrier()

        nj = (nchunks + NW - 1 - wid) // NW

        def body(j, carry):
            off0 = (wid + (2 * j) * NW) * C
            off1 = (wid + (2 * j + 1) * NW) * C
            pltpu.sync_copy(src_hbm.at[pl.ds(off0, C)], srcb0)
            pltpu.sync_copy(dst_hbm.at[pl.ds(off0, C)], dstb0)
            g0 = pltpu.async_copy(wy_hbm.at[srcb0], rows0, sem1)
            pltpu.sync_copy(src_hbm.at[pl.ds(off1, C)], srcb1)
            pltpu.sync_copy(dst_hbm.at[pl.ds(off1, C)], dstb1)
            g1 = pltpu.async_copy(wy_hbm.at[srcb1], rows1, sem2)
            g0.wait()
            pltpu.sync_copy(rows0, acc16.at[dstb0], add=True)
            g1.wait()
            pltpu.sync_copy(rows1, acc16.at[dstb1], add=True)
            return carry

        lax.fori_loop(0, nj // 2, body, 0)

        @pl.when(nj % 2 == 1)
        def _tail():
            off0 = (wid + (nj - 1) * NW) * C
            pltpu.sync_copy(src_hbm.at[pl.ds(off0, C)], srcb0)
            pltpu.sync_copy(dst_hbm.at[pl.ds(off0, C)], dstb0)
            pltpu.async_copy(wy_hbm.at[srcb0], rows0, sem1).wait()
            pltpu.sync_copy(rows0, acc16.at[dstb0], add=True)

        plsc.subcore_barrier()
        for j in range(rpt // C):
            r0 = zb + j * C
            pltpu.sync_copy(acc16.at[pl.ds(r0, C)], rows0)
            pltpu.sync_copy(rows0, out16_hbm.at[c, pl.ds(r0, C)])

    return push_kernel


_BN = 1000


def _grad_body(a_ref, z_ref, sel_ref, g_ref):
    zt = jnp.concatenate([z_ref[...]] * 16, axis=1)
    g_ref[...] = 2.0 * jnp.dot(a_ref[...] * zt, sel_ref[...],
                               preferred_element_type=jnp.float32,
                               precision=jax.lax.Precision.HIGHEST)


def _tc_grad(n):
    g = n // _BN
    return pl.pallas_call(
        _grad_body,
        grid=(g,),
        in_specs=[
            pl.BlockSpec((_BN, 256), lambda i: (i, 0)),
            pl.BlockSpec((_BN, 16), lambda i: (i, 0)),
            pl.BlockSpec((256, 16), lambda i: (0, 0)),
        ],
        out_specs=pl.BlockSpec((_BN, 16), lambda i: (i, 0)),
        out_shape=jax.ShapeDtypeStruct((n, 16), jnp.float32),
    )


def kernel(x, A, b, edge_index, num_layers):
    n, d = x.shape
    e = edge_index.shape[1]
    a2 = A.reshape(n, d * d)
    src = edge_index[0]
    dst = edge_index[1]
    sel = (jnp.arange(d * d)[:, None] // d ==
           jnp.arange(d)[None, :]).astype(jnp.float32)
    ones = jnp.ones((C,), jnp.float32)
    z16 = jnp.zeros((C, 16), jnp.float32)
    zv = jnp.zeros((ZRV,), jnp.float32)

    pdeg = _sc_degree(n, e)(src, ones, zv)
    inv = (1.0 / (pdeg[0, :n] + pdeg[1, :n] + 1.0))[:, None]
    pushqv = _sc_push16v(n, e)
    pushy = _sc_push16(n, e)
    grad = _tc_grad(n)

    y = grad(a2, x, sel) + b
    u = x
    v = jnp.ones((n, 1), x.dtype)
    x0 = x
    for k in range(4):
        wq = (u - STEP * y) * inv
        wy = y * inv
        wv = v * inv
        pq, pv = pushqv(wq, wv.reshape(n), src, dst, z16, zv)
        u = pq[0, :n] + pq[1, :n] + wq
        v = (pv[0, :n] + pv[1, :n])[:, None] + wv
        x1 = u / v
        if k < 3:
            py = pushy(wy, src, dst, z16)
            y = py[0, :n] + py[1, :n] + wy + grad(a2, x1 - x0, sel)
            x0 = x1
    comm = jnp.asarray(3 * e * num_layers, jnp.int32)
    return x1, comm
